# baseline (speedup 1.0000x reference)
import functools

import jax
import jax.numpy as jnp
from jax import lax
from jax.experimental import pallas as pl
from jax.experimental.pallas import tpu as pltpu
from jax.experimental.pallas import tpu_sc as plsc

B, S, D, E, DFF, CAP = 2, 2048, 768, 8, 3072, 320
N = B * S
RPE = B * CAP
NX = E * RPE
NXP = NX + 8
CH = 512
NBLK = N // CH
FK = 1536
RC = 512

_info = plsc.get_sparse_core_info()
_NC, _NS = _info.num_cores, _info.num_subcores
NW = _NC * _NS
RPW = N // NW


def _router_body(x_ref, wgt_ref, logits_ref, tp_ref, eidx_ref, dst_ref,
                 gidx_ref, cnt_ref):
    b = pl.program_id(0)
    j = pl.program_id(1)

    @pl.when(j == 0)
    def _():
        cnt_ref[...] = jnp.zeros_like(cnt_ref)

    x = x_ref[0].astype(jnp.bfloat16)
    logits = lax.dot_general(x, wgt_ref[...].astype(jnp.bfloat16),
                             (((1,), (0,)), ((), ())),
                             preferred_element_type=jnp.float32)
    logits_ref[0] = logits
    m = jnp.max(logits, axis=1, keepdims=True)
    p = jnp.exp(logits - m)
    s = jnp.sum(p, axis=1, keepdims=True)
    tp = 1.0 / s
    ecol = lax.broadcasted_iota(jnp.int32, (CH, E), 1)
    eidx = jnp.min(jnp.where(logits == m, ecol, E), axis=1, keepdims=True)
    one_hot = (ecol == eidx).astype(jnp.float32)
    rr = lax.broadcasted_iota(jnp.int32, (CH, CH), 0)
    cc = lax.broadcasted_iota(jnp.int32, (CH, CH), 1)
    tri = (rr >= cc).astype(jnp.bfloat16)
    csum = lax.dot_general(tri, one_hot.astype(jnp.bfloat16),
                           (((1,), (0,)), ((), ())),
                           preferred_element_type=jnp.float32)
    prio = csum + cnt_ref[...]
    cnt_ref[...] = cnt_ref[...] + jnp.sum(one_hot, axis=0, keepdims=True)
    tprio = jnp.sum(one_hot * prio, axis=1, keepdims=True)
    kept = tprio <= CAP
    slot = tprio.astype(jnp.int32) - 1
    base = eidx * RPE + b * CAP + slot
    dst_ref[0] = jnp.where(kept, base, NX)
    gidx_ref[0] = jnp.where(kept, base, 0)
    eidx_ref[0] = jnp.where(kept, eidx, 0)
    tp_ref[0] = jnp.where(kept, tp, -tp)


def _router(data, wgt):
    sj = S // CH
    return pl.pallas_call(
        _router_body,
        grid=(B, sj),
        in_specs=[
            pl.BlockSpec((1, CH, D), lambda b, j: (b, j, 0)),
            pl.BlockSpec((D, E), lambda b, j: (0, 0)),
        ],
        out_specs=[
            pl.BlockSpec((1, CH, E), lambda b, j: (b, j, 0)),
            pl.BlockSpec((1, CH, 1), lambda b, j: (b * sj + j, 0, 0)),
            pl.BlockSpec((1, CH, 1), lambda b, j: (b * sj + j, 0, 0)),
            pl.BlockSpec((1, CH, 1), lambda b, j: (b * sj + j, 0, 0)),
            pl.BlockSpec((1, CH, 1), lambda b, j: (b * sj + j, 0, 0)),
        ],
        out_shape=[
            jax.ShapeDtypeStruct((B, S, E), jnp.float32),
            jax.ShapeDtypeStruct((NBLK, CH, 1), jnp.float32),
            jax.ShapeDtypeStruct((NBLK, CH, 1), jnp.int32),
            jax.ShapeDtypeStruct((NBLK, CH, 1), jnp.int32),
            jax.ShapeDtypeStruct((NBLK, CH, 1), jnp.int32),
        ],
        scratch_shapes=[pltpu.VMEM((1, E), jnp.float32)],
    )(data, wgt)


_mesh = plsc.VectorSubcoreMesh(core_axis_name="c", subcore_axis_name="s")


HRW = RPW // 2


@functools.partial(
    pl.kernel, mesh=_mesh,
    out_type=jax.ShapeDtypeStruct((NXP, D), jnp.float32),
    scratch_types=[
        pltpu.VMEM((HRW,), jnp.int32),
        pltpu.VMEM((HRW,), jnp.int32),
        pltpu.VMEM((HRW, D), jnp.float32),
        pltpu.VMEM((HRW, D), jnp.float32),
        pltpu.SemaphoreType.DMA,
        pltpu.SemaphoreType.DMA,
        pltpu.SemaphoreType.DMA,
    ],
)
def _sc_scatter(data_hbm, dst_hbm, x_hbm, idx0_v, idx1_v, r0_v, r1_v,
                s0, s1, sw):
    wid = lax.axis_index("s") * _NC + lax.axis_index("c")
    base = wid * RPW
    pltpu.sync_copy(dst_hbm.at[pl.ds(base, HRW)], idx0_v)
    pltpu.sync_copy(dst_hbm.at[pl.ds(base + HRW, HRW)], idx1_v)
    c0 = pltpu.async_copy(data_hbm.at[pl.ds(base, HRW)], r0_v, s0)
    c1 = pltpu.async_copy(data_hbm.at[pl.ds(base + HRW, HRW)], r1_v, s1)
    c0.wait()
    w0 = pltpu.async_copy(r0_v, x_hbm.at[idx0_v], sw)
    c1.wait()
    w0.wait()
    pltpu.async_copy(r1_v, x_hbm.at[idx1_v], sw).wait()


def _ffn_body(x_ref, w1_ref, w2_ref, y_ref):
    h = jnp.maximum(
        lax.dot_general(x_ref[...], w1_ref[0], (((1,), (0,)), ((), ()))),
        0.0)
    y_ref[...] = lax.dot_general(h, w2_ref[0], (((1,), (0,)), ((), ())))


def _ffn(X, W1, W2):
    return pl.pallas_call(
        _ffn_body,
        grid=(E,),
        in_specs=[
            pl.BlockSpec((RPE, D), lambda e: (e, 0)),
            pl.BlockSpec((1, D, DFF), lambda e: (e, 0, 0)),
            pl.BlockSpec((1, DFF, D), lambda e: (e, 0, 0)),
        ],
        out_specs=pl.BlockSpec((RPE, D), lambda e: (e, 0)),
        out_shape=jax.ShapeDtypeStruct((NX, D), jnp.float32),
        compiler_params=pltpu.CompilerParams(
            dimension_semantics=("arbitrary",)),
    )(X, W1, W2)


@functools.partial(
    pl.kernel, mesh=_mesh,
    out_type=jax.ShapeDtypeStruct((N, D), jnp.float32),
    scratch_types=[
        pltpu.VMEM((HRW,), jnp.int32),
        pltpu.VMEM((HRW,), jnp.int32),
        pltpu.VMEM((HRW, D), jnp.float32),
        pltpu.VMEM((HRW, D), jnp.float32),
        pltpu.SemaphoreType.DMA,
        pltpu.SemaphoreType.DMA,
        pltpu.SemaphoreType.DMA,
    ],
)
def _sc_gather(y_hbm, gidx_hbm, out_hbm, idx0_v, idx1_v, r0_v, r1_v,
               s0, s1, sw):
    wid = lax.axis_index("s") * _NC + lax.axis_index("c")
    base = wid * RPW
    pltpu.sync_copy(gidx_hbm.at[pl.ds(base, HRW)], idx0_v)
    pltpu.sync_copy(gidx_hbm.at[pl.ds(base + HRW, HRW)], idx1_v)
    g0 = pltpu.async_copy(y_hbm.at[idx0_v], r0_v, s0)
    g1 = pltpu.async_copy(y_hbm.at[idx1_v], r1_v, s1)
    g0.wait()
    w0 = pltpu.async_copy(r0_v, out_hbm.at[pl.ds(base, HRW)], sw)
    g1.wait()
    w0.wait()
    pltpu.async_copy(r1_v, out_hbm.at[pl.ds(base + HRW, HRW)], sw).wait()


def _combine_body(yg_ref, d_ref, tpk_ref, out_ref):
    tpk = tpk_ref[0]
    keep = tpk > 0.0
    out_ref[...] = jnp.abs(tpk) * jnp.where(keep, yg_ref[...], d_ref[...])


def _combine(Yg, data2, tpk):
    return pl.pallas_call(
        _combine_body,
        grid=(N // RC,),
        in_specs=[
            pl.BlockSpec((RC, D), lambda i: (i, 0)),
            pl.BlockSpec((RC, D), lambda i: (i, 0)),
            pl.BlockSpec((1, RC, 1), lambda i: (i, 0, 0)),
        ],
        out_specs=pl.BlockSpec((RC, D), lambda i: (i, 0)),
        out_shape=jax.ShapeDtypeStruct((N, D), jnp.float32),
    )(Yg, data2, tpk)


def kernel(data, Wg, W1, W2):
    data2 = data.reshape(N, D)
    wgt = Wg.T
    logits, tpk, eidx, dst, gidx = _router(data, wgt)
    X = _sc_scatter(data2, dst.reshape(N))
    Y = _ffn(X, W1, W2)
    Yg = _sc_gather(Y, gidx.reshape(N))
    hid = _combine(Yg, data2, tpk)
    return hid.reshape(B, S, D), logits, eidx.reshape(B, S)


# speedup vs baseline: 1.0086x; 1.0086x over previous
import functools

import jax
import jax.numpy as jnp
from jax import lax
from jax.experimental import pallas as pl
from jax.experimental.pallas import tpu as pltpu
from jax.experimental.pallas import tpu_sc as plsc

B, S, D, E, DFF, CAP = 2, 2048, 768, 8, 3072, 320
N = B * S
RPE = B * CAP
NX = E * RPE
NXP = NX + 8
CH = 512
NBLK = N // CH
FK = 1536
RC = 2048

_info = plsc.get_sparse_core_info()
_NC, _NS = _info.num_cores, _info.num_subcores
NW = _NC * _NS
RPW = N // NW


def _router_body(x_ref, wgt_ref, tri_ref, logits_ref, tp_ref, eidx_ref,
                 dst_ref, gidx_ref, cnt_ref):
    b = pl.program_id(0)
    j = pl.program_id(1)

    @pl.when(j == 0)
    def _():
        cnt_ref[...] = jnp.zeros_like(cnt_ref)

    x = x_ref[0].astype(jnp.bfloat16)
    logits = lax.dot_general(x, wgt_ref[...].astype(jnp.bfloat16),
                             (((1,), (0,)), ((), ())),
                             preferred_element_type=jnp.float32)
    logits_ref[0] = logits
    m = jnp.max(logits, axis=1, keepdims=True)
    p = jnp.exp(logits - m)
    s = jnp.sum(p, axis=1, keepdims=True)
    tp = 1.0 / s
    ecol = lax.broadcasted_iota(jnp.int32, (CH, E), 1)
    eidx = jnp.min(jnp.where(logits == m, ecol, E), axis=1, keepdims=True)
    one_hot = (ecol == eidx).astype(jnp.float32)
    csum = lax.dot_general(tri_ref[...], one_hot.astype(jnp.bfloat16),
                           (((1,), (0,)), ((), ())),
                           preferred_element_type=jnp.float32)
    prio = csum + cnt_ref[...]
    cnt_ref[...] = cnt_ref[...] + jnp.sum(one_hot, axis=0, keepdims=True)
    tprio = jnp.sum(one_hot * prio, axis=1, keepdims=True)
    kept = tprio <= CAP
    slot = tprio.astype(jnp.int32) - 1
    base = eidx * RPE + b * CAP + slot
    dst_ref[0] = jnp.where(kept, base, NX)
    gidx_ref[0] = jnp.where(kept, base, 0)
    eidx_ref[0] = jnp.where(kept, eidx, 0)
    tp_ref[0] = jnp.where(kept, tp, -tp)


def _router(data, wgt, tri):
    sj = S // CH
    return pl.pallas_call(
        _router_body,
        grid=(B, sj),
        in_specs=[
            pl.BlockSpec((1, CH, D), lambda b, j: (b, j, 0)),
            pl.BlockSpec((D, E), lambda b, j: (0, 0)),
            pl.BlockSpec((CH, CH), lambda b, j: (0, 0)),
        ],
        out_specs=[
            pl.BlockSpec((1, CH, E), lambda b, j: (b, j, 0)),
            pl.BlockSpec((1, CH, 1), lambda b, j: (b * sj + j, 0, 0)),
            pl.BlockSpec((1, CH, 1), lambda b, j: (b * sj + j, 0, 0)),
            pl.BlockSpec((1, CH, 1), lambda b, j: (b * sj + j, 0, 0)),
            pl.BlockSpec((1, CH, 1), lambda b, j: (b * sj + j, 0, 0)),
        ],
        out_shape=[
            jax.ShapeDtypeStruct((B, S, E), jnp.float32),
            jax.ShapeDtypeStruct((NBLK, CH, 1), jnp.float32),
            jax.ShapeDtypeStruct((NBLK, CH, 1), jnp.int32),
            jax.ShapeDtypeStruct((NBLK, CH, 1), jnp.int32),
            jax.ShapeDtypeStruct((NBLK, CH, 1), jnp.int32),
        ],
        scratch_shapes=[pltpu.VMEM((1, E), jnp.float32)],
    )(data, wgt, tri)


_mesh = plsc.VectorSubcoreMesh(core_axis_name="c", subcore_axis_name="s")


@functools.partial(
    pl.kernel, mesh=_mesh,
    out_type=jax.ShapeDtypeStruct((NXP, D), jnp.float32),
    scratch_types=[
        pltpu.VMEM((RPW,), jnp.int32),
        pltpu.VMEM((RPW, D), jnp.float32),
        pltpu.SemaphoreType.DMA,
    ],
)
def _sc_scatter(data_hbm, dst_hbm, x_hbm, idx_v, rows_v, sem):
    wid = lax.axis_index("s") * _NC + lax.axis_index("c")
    base = wid * RPW
    pltpu.sync_copy(dst_hbm.at[pl.ds(base, RPW)], idx_v)
    pltpu.sync_copy(data_hbm.at[pl.ds(base, RPW)], rows_v)
    pltpu.async_copy(rows_v, x_hbm.at[idx_v], sem).wait()


def _ffn_body(x_ref, w1_ref, w2_ref, y_ref):
    h = jnp.maximum(
        lax.dot_general(x_ref[...], w1_ref[0], (((1,), (0,)), ((), ()))),
        0.0)
    y_ref[...] = lax.dot_general(h, w2_ref[0], (((1,), (0,)), ((), ())))


def _ffn(X, W1, W2):
    return pl.pallas_call(
        _ffn_body,
        grid=(E,),
        in_specs=[
            pl.BlockSpec((RPE, D), lambda e: (e, 0)),
            pl.BlockSpec((1, D, DFF), lambda e: (e, 0, 0)),
            pl.BlockSpec((1, DFF, D), lambda e: (e, 0, 0)),
        ],
        out_specs=pl.BlockSpec((RPE, D), lambda e: (e, 0)),
        out_shape=jax.ShapeDtypeStruct((NX, D), jnp.float32),
        compiler_params=pltpu.CompilerParams(
            dimension_semantics=("arbitrary",)),
    )(X, W1, W2)


@functools.partial(
    pl.kernel, mesh=_mesh,
    out_type=jax.ShapeDtypeStruct((N, D), jnp.float32),
    scratch_types=[
        pltpu.VMEM((RPW,), jnp.int32),
        pltpu.VMEM((RPW, D), jnp.float32),
        pltpu.SemaphoreType.DMA,
    ],
)
def _sc_gather(y_hbm, gidx_hbm, out_hbm, idx_v, rows_v, sem):
    wid = lax.axis_index("s") * _NC + lax.axis_index("c")
    base = wid * RPW
    pltpu.sync_copy(gidx_hbm.at[pl.ds(base, RPW)], idx_v)
    pltpu.async_copy(y_hbm.at[idx_v], rows_v, sem).wait()
    pltpu.sync_copy(rows_v, out_hbm.at[pl.ds(base, RPW)])


def _combine_body(yg_ref, d_ref, tpk_ref, out_ref):
    tpk = tpk_ref[0]
    keep = tpk > 0.0
    out_ref[...] = jnp.abs(tpk) * jnp.where(keep, yg_ref[...], d_ref[...])


def _combine(Yg, data2, tpk):
    return pl.pallas_call(
        _combine_body,
        grid=(N // RC,),
        in_specs=[
            pl.BlockSpec((RC, D), lambda i: (i, 0)),
            pl.BlockSpec((RC, D), lambda i: (i, 0)),
            pl.BlockSpec((1, RC, 1), lambda i: (i, 0, 0)),
        ],
        out_specs=pl.BlockSpec((RC, D), lambda i: (i, 0)),
        out_shape=jax.ShapeDtypeStruct((N, D), jnp.float32),
    )(Yg, data2, tpk)


def kernel(data, Wg, W1, W2):
    data2 = data.reshape(N, D)
    wgt = Wg.T
    tri = jnp.tril(jnp.ones((CH, CH), jnp.bfloat16))
    logits, tpk, eidx, dst, gidx = _router(data, wgt, tri)
    X = _sc_scatter(data2, dst.reshape(N))
    Y = _ffn(X, W1, W2)
    Yg = _sc_gather(Y, gidx.reshape(N))
    hid = _combine(Yg, data2, tpk)
    return hid.reshape(B, S, D), logits, eidx.reshape(B, S)
